# baseline (speedup 1.0000x reference)
import functools
import math

import jax
import jax.numpy as jnp
from jax import lax
from jax.experimental import pallas as pl
from jax.experimental.pallas import tpu as pltpu
from jax.experimental.pallas import tpu_sc as plsc

N = 10000
E = 160000
DIM = 128
NR = 6
NL = 6
CUT = 5.0
NG = 256
P = 5

N_PAD = 10240
E_PAD = 163840
NW = 32
EPW = E_PAD // NW
C = 64
NCH = EPW // C
BN = 1024
BE = 2048


def _silu(v):
    return v * (1.0 / (1.0 + jnp.exp(-v)))


def _d2_kernel(px_h, py_h, pz_h, row_h, col_h, d2_h,
               pxv, pyv, pzv, rv, cv, dbuf):
    cid = lax.axis_index("c")
    sid = lax.axis_index("s")
    wid = sid * 2 + cid
    pltpu.sync_copy(px_h, pxv)
    pltpu.sync_copy(py_h, pyv)
    pltpu.sync_copy(pz_h, pzv)
    C2 = 512

    def chunk(g, _):
        base = wid * EPW + g * C2
        pltpu.sync_copy(row_h.at[pl.ds(base, C2)], rv)
        pltpu.sync_copy(col_h.at[pl.ds(base, C2)], cv)

        def vec(i, _):
            r16 = rv[pl.ds(i * 16, 16)]
            c16 = cv[pl.ds(i * 16, 16)]
            dx = plsc.load_gather(pxv, [r16]) - plsc.load_gather(pxv, [c16])
            dy = plsc.load_gather(pyv, [r16]) - plsc.load_gather(pyv, [c16])
            dz = plsc.load_gather(pzv, [r16]) - plsc.load_gather(pzv, [c16])
            dbuf[pl.ds(i * 16, 16)] = dx * dx + dy * dy + dz * dz
            return 0

        lax.fori_loop(0, C2 // 16, vec, 0, unroll=4)
        pltpu.sync_copy(dbuf, d2_h.at[pl.ds(base, C2)])
        return 0

    lax.fori_loop(0, EPW // C2, chunk, 0)


def _d2_sc(px, py, pz, rows, cols):
    mesh = plsc.VectorSubcoreMesh(core_axis_name="c", subcore_axis_name="s")
    return pl.kernel(
        _d2_kernel,
        out_type=jax.ShapeDtypeStruct((E_PAD,), jnp.float32),
        mesh=mesh,
        compiler_params=pltpu.CompilerParams(needs_layout_passes=False),
        scratch_types=[
            pltpu.VMEM((N_PAD,), jnp.float32),
            pltpu.VMEM((N_PAD,), jnp.float32),
            pltpu.VMEM((N_PAD,), jnp.float32),
            pltpu.VMEM((512,), jnp.int32),
            pltpu.VMEM((512,), jnp.int32),
            pltpu.VMEM((512,), jnp.float32),
        ],
    )(px, py, pz, rows, cols)


def _edge_kernel(a_h, b_h, ec_h, le_h, row_h, col_h, out_h,
                 rv, cv, ga, gb, ecv, lev, agg, sem1, sem2):
    cid = lax.axis_index("c")
    sid = lax.axis_index("s")
    wid = sid * 2 + cid

    def zrow(r, _):
        for j in range(8):
            ga[r, pl.ds(j * 16, 16)] = jnp.zeros((16,), jnp.float32)
        return 0

    lax.fori_loop(0, C, zrow, 0)

    def zcopy(k, _):
        pltpu.sync_copy(ga, agg.at[pl.ds(sid * 640 + k * C, C), :])
        return 0

    lax.fori_loop(0, N_PAD // 16 // C, zcopy, 0)
    plsc.subcore_barrier()

    def chunk(g, _):
        base = wid * EPW + g * C
        pltpu.sync_copy(row_h.at[pl.ds(base, C)], rv)
        pltpu.sync_copy(col_h.at[pl.ds(base, C)], cv)
        cp1 = pltpu.async_copy(a_h.at[rv], ga, sem1)
        cp2 = pltpu.async_copy(b_h.at[cv], gb, sem2)
        pltpu.sync_copy(ec_h.at[pl.ds(base, C), :], ecv)
        pltpu.sync_copy(le_h.at[pl.ds(base, C), :], lev)
        cp1.wait()
        cp2.wait()

        def row(r, _):
            for j in range(8):
                sl = pl.ds(j * 16, 16)
                s = ga[r, sl] + gb[r, sl] + ecv[r, sl]
                x = s * (1.0 / (1.0 + jnp.exp(-s)))
                lev[r, sl] = lev[r, sl] * x
            return 0

        lax.fori_loop(0, C, row, 0, unroll=2)
        pltpu.sync_copy(lev, agg.at[rv], add=True)
        return 0

    lax.fori_loop(0, NCH, chunk, 0)
    plsc.subcore_barrier()

    def dump(k, _):
        off = sid * 640 + k * C
        pltpu.sync_copy(agg.at[pl.ds(off, C), :],
                        out_h.at[pl.ds(cid * N_PAD + off, C), :])
        return 0

    lax.fori_loop(0, N_PAD // 16 // C, dump, 0)


def _edge_sc(a, b, ec, le, rows, cols):
    mesh = plsc.VectorSubcoreMesh(core_axis_name="c", subcore_axis_name="s")
    out = pl.kernel(
        _edge_kernel,
        out_type=jax.ShapeDtypeStruct((2 * N_PAD, DIM), jnp.float32),
        mesh=mesh,
        compiler_params=pltpu.CompilerParams(needs_layout_passes=False),
        scratch_types=[
            pltpu.VMEM((C,), jnp.int32),
            pltpu.VMEM((C,), jnp.int32),
            pltpu.VMEM((C, DIM), jnp.float32),
            pltpu.VMEM((C, DIM), jnp.float32),
            pltpu.VMEM((C, DIM), jnp.float32),
            pltpu.VMEM((C, DIM), jnp.float32),
            pltpu.VMEM_SHARED((N_PAD, DIM), jnp.float32),
            pltpu.SemaphoreType.DMA,
            pltpu.SemaphoreType.DMA,
        ],
    )(a, b, ec, le, rows, cols)
    return out[:N_PAD], out[N_PAD:]


def _eale_body(d2_ref, freq_ref, rbfW_ref, rbfb_ref, Wc_ref, lin_ref, xeb_ref,
               ec_ref, le_ref, ea_ref):
    l = pl.program_id(1)

    @pl.when(l == 0)
    def _():
        v = jnp.sqrt(d2_ref[...] + 1e-12) / CUT
        xp = v * v * v * v * v
        a = -(P + 1) * (P + 2) / 2.0
        b = P * (P + 2)
        c = -P * (P + 1) / 2.0
        env = 1.0 / v + a * xp + b * xp * v + c * xp * v * v
        env = jnp.where(v < 1.0, env, jnp.zeros_like(v))
        rbf = env * jnp.sin(freq_ref[...] * v)
        ea = _silu(jnp.dot(rbf, rbfW_ref[...],
                           preferred_element_type=jnp.float32) + rbfb_ref[...])
        ea_ref[...] = ea

    ea = ea_ref[...]
    ec_ref[0] = jnp.dot(ea, Wc_ref[0],
                        preferred_element_type=jnp.float32) + xeb_ref[0]
    le_ref[0] = jnp.dot(ea, lin_ref[0],
                        preferred_element_type=jnp.float32)


def _eale_tc(d2, freqp, rbfW8, rbf_b, Wc_all, lin_all, xe_b):
    grid = (E_PAD // BE, NL)
    return pl.pallas_call(
        _eale_body,
        grid=grid,
        in_specs=[
            pl.BlockSpec((BE, 1), lambda e, l: (e, 0)),
            pl.BlockSpec((1, 8), lambda e, l: (0, 0)),
            pl.BlockSpec((8, DIM), lambda e, l: (0, 0)),
            pl.BlockSpec((1, DIM), lambda e, l: (0, 0)),
            pl.BlockSpec((1, DIM, DIM), lambda e, l: (l, 0, 0)),
            pl.BlockSpec((1, DIM, DIM), lambda e, l: (l, 0, 0)),
            pl.BlockSpec((1, 1, DIM), lambda e, l: (l, 0, 0)),
        ],
        out_specs=[
            pl.BlockSpec((1, BE, DIM), lambda e, l: (l, e, 0)),
            pl.BlockSpec((1, BE, DIM), lambda e, l: (l, e, 0)),
        ],
        out_shape=[
            jax.ShapeDtypeStruct((NL, E_PAD, DIM), jnp.float32),
            jax.ShapeDtypeStruct((NL, E_PAD, DIM), jnp.float32),
        ],
        scratch_shapes=[pltpu.VMEM((BE, DIM), jnp.float32)],
    )(d2, freqp, rbfW8, rbf_b, Wc_all, lin_all, xe_b)


def _apart(h, hW_ref, hb_ref, Wa_ref, Wb_ref):
    hh = _silu(jnp.dot(h, hW_ref[0], preferred_element_type=jnp.float32)
               + hb_ref[...])
    a = jnp.dot(hh, Wa_ref[0], preferred_element_type=jnp.float32)
    b = jnp.dot(hh, Wb_ref[0], preferred_element_type=jnp.float32)
    return hh, a, b


def _node0_body(xf_ref, embp_ref, hW_ref, hb_ref, Wa_ref, Wb_ref,
                h_ref, hh_ref, a_ref, b_ref):
    oh = (xf_ref[...] ==
          lax.broadcasted_iota(jnp.int32, (1, 8), 1).astype(jnp.float32))
    h = jnp.dot(oh.astype(jnp.float32), embp_ref[...],
                preferred_element_type=jnp.float32)
    h_ref[...] = h
    hh, a, b = _apart(h, hW_ref, hb_ref, Wa_ref, Wb_ref)
    hh_ref[...] = hh
    a_ref[...] = a
    b_ref[...] = b


def _node0_tc(xf, embp, hW, hb, Wa, Wb):
    nspec = pl.BlockSpec((BN, DIM), lambda i: (i, 0))
    wspec = pl.BlockSpec((1, DIM, DIM), lambda i: (0, 0, 0))
    bspec = pl.BlockSpec((1, DIM), lambda i: (0, 0))
    return pl.pallas_call(
        _node0_body,
        grid=(N_PAD // BN,),
        in_specs=[
            pl.BlockSpec((BN, 1), lambda i: (i, 0)),
            pl.BlockSpec((8, DIM), lambda i: (0, 0)),
            wspec, bspec, wspec, wspec,
        ],
        out_specs=[nspec, nspec, nspec, nspec],
        out_shape=[jax.ShapeDtypeStruct((N_PAD, DIM), jnp.float32)] * 4,
    )(xf, embp, hW, hb, Wa, Wb)


def _bpart(p0, p1, hh, hres, y, resW_ref, resb_ref, mW_ref, mb_ref,
           oW_ref, ob_ref):
    t = p0 + p1 + hh + hres
    u = _silu(jnp.dot(t, resW_ref[0, 0, 0], preferred_element_type=jnp.float32)
              + resb_ref[0, 0, 0][None])
    t = t + _silu(jnp.dot(u, resW_ref[0, 0, 1], preferred_element_type=jnp.float32)
                  + resb_ref[0, 0, 1][None])
    h2 = _silu(jnp.dot(t, mW_ref[0], preferred_element_type=jnp.float32)
               + mb_ref[...]) + hres
    u = _silu(jnp.dot(h2, resW_ref[0, 1, 0], preferred_element_type=jnp.float32)
              + resb_ref[0, 1, 0][None])
    h2 = h2 + _silu(jnp.dot(u, resW_ref[0, 1, 1], preferred_element_type=jnp.float32)
                    + resb_ref[0, 1, 1][None])
    u = _silu(jnp.dot(h2, resW_ref[0, 2, 0], preferred_element_type=jnp.float32)
              + resb_ref[0, 2, 0][None])
    h2 = h2 + _silu(jnp.dot(u, resW_ref[0, 2, 1], preferred_element_type=jnp.float32)
                    + resb_ref[0, 2, 1][None])
    ynew = y + jnp.dot(h2, oW_ref[0], preferred_element_type=jnp.float32) \
        + ob_ref[...]
    return h2, ynew


def _nodeB_body(p0_ref, p1_ref, hh_ref, hres_ref, y_ref,
                resW_ref, resb_ref, mW_ref, mb_ref, oW_ref, ob_ref,
                h_ref, yout_ref):
    h2, ynew = _bpart(p0_ref[...], p1_ref[...], hh_ref[...], hres_ref[...],
                      y_ref[...], resW_ref, resb_ref, mW_ref, mb_ref,
                      oW_ref, ob_ref)
    h_ref[...] = h2
    yout_ref[...] = ynew


def _nodeBA_body(p0_ref, p1_ref, hh_ref, hres_ref, y_ref,
                 resW_ref, resb_ref, mW_ref, mb_ref, oW_ref, ob_ref,
                 hW_ref, hb_ref, Wa_ref, Wb_ref,
                 h_ref, yout_ref, hh2_ref, a_ref, b_ref):
    h2, ynew = _bpart(p0_ref[...], p1_ref[...], hh_ref[...], hres_ref[...],
                      y_ref[...], resW_ref, resb_ref, mW_ref, mb_ref,
                      oW_ref, ob_ref)
    h_ref[...] = h2
    yout_ref[...] = ynew
    hh, a, b = _apart(h2, hW_ref, hb_ref, Wa_ref, Wb_ref)
    hh2_ref[...] = hh
    a_ref[...] = a
    b_ref[...] = b


def _node_tc(p0, p1, hh, hres, y, resW, resb, mW, mb, oW, ob,
             nxt=None):
    nspec = pl.BlockSpec((BN, DIM), lambda i: (i, 0))
    yspec = pl.BlockSpec((BN, 1), lambda i: (i, 0))
    wspec = pl.BlockSpec((1, DIM, DIM), lambda i: (0, 0, 0))
    bspec = pl.BlockSpec((1, DIM), lambda i: (0, 0))
    in_specs = [
        nspec, nspec, nspec, nspec, yspec,
        pl.BlockSpec((1, 3, 2, DIM, DIM), lambda i: (0, 0, 0, 0, 0)),
        pl.BlockSpec((1, 3, 2, DIM), lambda i: (0, 0, 0, 0)),
        wspec, bspec,
        pl.BlockSpec((1, DIM, 1), lambda i: (0, 0, 0)),
        pl.BlockSpec((1, 1), lambda i: (0, 0)),
    ]
    args = [p0, p1, hh, hres, y, resW, resb, mW, mb, oW, ob]
    if nxt is None:
        return pl.pallas_call(
            _nodeB_body,
            grid=(N_PAD // BN,),
            in_specs=in_specs,
            out_specs=[nspec, yspec],
            out_shape=[jax.ShapeDtypeStruct((N_PAD, DIM), jnp.float32),
                       jax.ShapeDtypeStruct((N_PAD, 1), jnp.float32)],
        )(*args)
    hW, hb, Wa, Wb = nxt
    return pl.pallas_call(
        _nodeBA_body,
        grid=(N_PAD // BN,),
        in_specs=in_specs + [wspec, bspec, wspec, wspec],
        out_specs=[nspec, yspec, nspec, nspec, nspec],
        out_shape=[jax.ShapeDtypeStruct((N_PAD, DIM), jnp.float32),
                   jax.ShapeDtypeStruct((N_PAD, 1), jnp.float32),
                   jax.ShapeDtypeStruct((N_PAD, DIM), jnp.float32),
                   jax.ShapeDtypeStruct((N_PAD, DIM), jnp.float32),
                   jax.ShapeDtypeStruct((N_PAD, DIM), jnp.float32)],
    )(*args, hW, hb, Wa, Wb)


def _pool_body(y_ref, bf_ref, out_ref):
    i = pl.program_id(0)

    @pl.when(i == 0)
    def _():
        out_ref[...] = jnp.zeros_like(out_ref)

    oh = (bf_ref[...] ==
          lax.broadcasted_iota(jnp.int32, (1, NG), 1).astype(jnp.float32))
    out_ref[...] += jnp.sum(oh.astype(jnp.float32) * y_ref[...], axis=0,
                            keepdims=True)


def _pool_tc(y, bf):
    return pl.pallas_call(
        _pool_body,
        grid=(N_PAD // BN,),
        in_specs=[pl.BlockSpec((BN, 1), lambda i: (i, 0)),
                  pl.BlockSpec((BN, 1), lambda i: (i, 0))],
        out_specs=pl.BlockSpec((1, NG), lambda i: (0, 0)),
        out_shape=jax.ShapeDtypeStruct((1, NG), jnp.float32),
    )(y, bf)


def kernel(x, pos, edge_index, batch, emb, freq, rbf_W, rbf_b, h_W, h_b,
           xe_W, xe_b, lin_W, res_W, res_b, mlp_W, mlp_b, out_W, out_b):
    f32 = jnp.float32
    rows = jnp.concatenate(
        [edge_index[0].astype(jnp.int32),
         jnp.full((E_PAD - E,), N, jnp.int32)])
    cols = jnp.concatenate(
        [edge_index[1].astype(jnp.int32),
         jnp.full((E_PAD - E,), N, jnp.int32)])
    posp = jnp.concatenate([pos.astype(f32),
                            jnp.zeros((N_PAD - N, 3), f32)], axis=0)
    px, py, pz = posp[:, 0], posp[:, 1], posp[:, 2]
    xf = jnp.concatenate([x.astype(f32), jnp.zeros((N_PAD - N,), f32)]
                         ).reshape(N_PAD, 1)
    bf = jnp.concatenate([batch.astype(f32),
                          jnp.full((N_PAD - N,), float(NG), f32)]
                         ).reshape(N_PAD, 1)

    embp = jnp.concatenate([emb, jnp.zeros((3, DIM), f32)], axis=0)
    freqp = jnp.concatenate([freq, jnp.zeros((2,), f32)]).reshape(1, 8)
    rbfW8 = jnp.concatenate([rbf_W, jnp.zeros((2, DIM), f32)], axis=0)
    rbfb2 = rbf_b.reshape(1, DIM)
    Wa = xe_W[:, :DIM]
    Wb = xe_W[:, DIM:2 * DIM]
    Wc = xe_W[:, 2 * DIM:]
    hb2 = h_b.reshape(NL, 1, DIM)
    mb2 = mlp_b.reshape(NL, 1, DIM)
    ob2 = out_b.reshape(NL, 1, 1)

    d2 = _d2_sc(px, py, pz, rows, cols)
    ec_all, le_all = _eale_tc(d2.reshape(E_PAD, 1), freqp, rbfW8, rbfb2,
                              Wc, lin_W, xe_b.reshape(NL, 1, DIM))

    h, hh, a, b = _node0_tc(xf, embp, h_W[0:1], hb2[0], Wa[0:1], Wb[0:1])
    y = jnp.zeros((N_PAD, 1), f32)
    for l in range(NL):
        p0, p1 = _edge_sc(a, b, ec_all[l], le_all[l], rows, cols)
        nxt = None if l == NL - 1 else (h_W[l + 1:l + 2], hb2[l + 1],
                                        Wa[l + 1:l + 2], Wb[l + 1:l + 2])
        outs = _node_tc(p0, p1, hh, h, y,
                        res_W[l:l + 1], res_b[l:l + 1], mlp_W[l:l + 1],
                        mb2[l], out_W[l:l + 1], ob2[l], nxt=nxt)
        if l == NL - 1:
            h, y = outs
        else:
            h, y, hh, a, b = outs

    pooled = _pool_tc(y, bf)
    return pooled.reshape(NG, 1)


# speedup vs baseline: 1.2479x; 1.2479x over previous
import functools
import math

import jax
import jax.numpy as jnp
from jax import lax
from jax.experimental import pallas as pl
from jax.experimental.pallas import tpu as pltpu
from jax.experimental.pallas import tpu_sc as plsc

N = 10000
E = 160000
DIM = 128
NR = 6
NL = 6
CUT = 5.0
NG = 256
P = 5

N_PAD = 10240
E_PAD = 163840
NW = 32
EPW = E_PAD // NW
C = 32
NCH = EPW // C
BN = 1024
BE = 2048


def _silu(v):
    return v * (1.0 / (1.0 + jnp.exp(-v)))


def _d2_kernel(px_h, py_h, pz_h, row_h, col_h, d2_h,
               pxv, pyv, pzv, rv, cv, dbuf):
    cid = lax.axis_index("c")
    sid = lax.axis_index("s")
    wid = sid * 2 + cid
    pltpu.sync_copy(px_h, pxv)
    pltpu.sync_copy(py_h, pyv)
    pltpu.sync_copy(pz_h, pzv)
    base = wid * EPW
    pltpu.sync_copy(row_h.at[pl.ds(base, EPW)], rv)
    pltpu.sync_copy(col_h.at[pl.ds(base, EPW)], cv)

    def vec(i, _):
        r16 = rv[pl.ds(i * 16, 16)]
        c16 = cv[pl.ds(i * 16, 16)]
        dx = plsc.load_gather(pxv, [r16]) - plsc.load_gather(pxv, [c16])
        dy = plsc.load_gather(pyv, [r16]) - plsc.load_gather(pyv, [c16])
        dz = plsc.load_gather(pzv, [r16]) - plsc.load_gather(pzv, [c16])
        dbuf[pl.ds(i * 16, 16)] = dx * dx + dy * dy + dz * dz
        return 0

    lax.fori_loop(0, EPW // 16, vec, 0, unroll=8)
    pltpu.sync_copy(dbuf, d2_h.at[pl.ds(base, EPW)])


def _d2_sc(px, py, pz, rows, cols):
    mesh = plsc.VectorSubcoreMesh(core_axis_name="c", subcore_axis_name="s")
    return pl.kernel(
        _d2_kernel,
        out_type=jax.ShapeDtypeStruct((E_PAD,), jnp.float32),
        mesh=mesh,
        compiler_params=pltpu.CompilerParams(needs_layout_passes=False),
        scratch_types=[
            pltpu.VMEM((N_PAD,), jnp.float32),
            pltpu.VMEM((N_PAD,), jnp.float32),
            pltpu.VMEM((N_PAD,), jnp.float32),
            pltpu.VMEM((EPW,), jnp.int32),
            pltpu.VMEM((EPW,), jnp.int32),
            pltpu.VMEM((EPW,), jnp.float32),
        ],
    )(px, py, pz, rows, cols)


def _edge_kernel(a_h, b_h, el_h, row_h, col_h, out_h,
                 rows_v, cols_v, ga, gb, el, msg, agg,
                 sem_in, sem_m):
    cid = lax.axis_index("c")
    sid = lax.axis_index("s")
    wid = sid * 2 + cid

    pltpu.sync_copy(row_h.at[pl.ds(wid * EPW, EPW)], rows_v)
    pltpu.sync_copy(col_h.at[pl.ds(wid * EPW, EPW)], cols_v)

    def zrow(r, _):
        for j in range(8):
            msg[0][r, pl.ds(j * 16, 16)] = jnp.zeros((16,), jnp.float32)
        return 0

    lax.fori_loop(0, C, zrow, 0)

    def zcopy(k, _):
        pltpu.sync_copy(msg[0], agg.at[pl.ds(sid * (N_PAD // 16) + k * C, C), :])
        return 0

    lax.fori_loop(0, N_PAD // 16 // C, zcopy, 0)
    plsc.subcore_barrier()

    def issue_in(g, s):
        base = wid * EPW + g * C
        pltpu.async_copy(a_h.at[rows_v.at[pl.ds(g * C, C)]], ga[s], sem_in[s])
        pltpu.async_copy(b_h.at[cols_v.at[pl.ds(g * C, C)]], gb[s], sem_in[s])
        pltpu.async_copy(el_h.at[pl.ds(base, C), :], el[s], sem_in[s])

    def drain_in(s):
        pltpu.make_async_copy(a_h.at[pl.ds(0, C), :], ga[s], sem_in[s]).wait()
        pltpu.make_async_copy(a_h.at[pl.ds(0, C), :], gb[s], sem_in[s]).wait()
        pltpu.make_async_copy(el_h.at[pl.ds(0, C), :], el[s],
                              sem_in[s]).wait()

    def drain_sc(s):
        pltpu.make_async_copy(a_h.at[pl.ds(0, C), :], msg[s], sem_m[s]).wait()

    def compute(s):
        def row(r, _):
            for j in range(8):
                sl = pl.ds(j * 16, 16)
                w = el[s][r, sl]
                ec16 = plsc.bitcast(w << 16, jnp.float32)
                le16 = plsc.bitcast(w & jnp.int32(-65536), jnp.float32)
                v = ga[s][r, sl] + gb[s][r, sl] + ec16
                x = v * (1.0 / (1.0 + jnp.exp(-v)))
                msg[s][r, sl] = le16 * x
            return 0

        lax.fori_loop(0, C, row, 0, unroll=2)

    issue_in(0, 0)
    issue_in(1, 1)

    def body(g, s):
        drain_in(s)

        @pl.when(g >= 2)
        def _():
            drain_sc(s)

        compute(s)
        for t in range(C // 16):
            rvec = rows_v[pl.ds(g * C + t * 16, 16)]
            pltpu.async_copy(msg[s].at[pl.ds(t * 16, 16), :],
                             agg.at[rvec], sem_m[s], add=True)

        @pl.when(g + 2 < NCH)
        def _():
            issue_in(g + 2, s)

    def pair(p, _):
        body(2 * p, 0)
        body(2 * p + 1, 1)
        return 0

    lax.fori_loop(0, NCH // 2, pair, 0)
    drain_sc(0)
    drain_sc(1)
    plsc.subcore_barrier()

    def dump(k, _):
        off = sid * (N_PAD // 16) + k * C
        pltpu.sync_copy(agg.at[pl.ds(off, C), :],
                        out_h.at[pl.ds(cid * N_PAD + off, C), :])
        return 0

    lax.fori_loop(0, N_PAD // 16 // C, dump, 0)


def _edge_sc(a, b, el, rows, cols):
    mesh = plsc.VectorSubcoreMesh(core_axis_name="c", subcore_axis_name="s")
    out = pl.kernel(
        _edge_kernel,
        out_type=jax.ShapeDtypeStruct((2 * N_PAD, DIM), jnp.float32),
        mesh=mesh,
        compiler_params=pltpu.CompilerParams(needs_layout_passes=False),
        scratch_types=[
            pltpu.VMEM((EPW,), jnp.int32),
            pltpu.VMEM((EPW,), jnp.int32),
            [pltpu.VMEM((C, DIM), jnp.float32)] * 2,
            [pltpu.VMEM((C, DIM), jnp.float32)] * 2,
            [pltpu.VMEM((C, DIM), jnp.int32)] * 2,
            [pltpu.VMEM((C, DIM), jnp.float32)] * 2,
            pltpu.VMEM_SHARED((N_PAD, DIM), jnp.float32),
            [pltpu.SemaphoreType.DMA] * 2,
            [pltpu.SemaphoreType.DMA] * 2,
        ],
    )(a, b, el, rows, cols)
    return out[:N_PAD], out[N_PAD:]


def _eale_body(d2_ref, freq_ref, rbfW_ref, rbfb_ref, Wc_ref, lin_ref, xeb_ref,
               el_ref, ea_ref):
    l = pl.program_id(1)

    @pl.when(l == 0)
    def _():
        v = jnp.sqrt(d2_ref[...] + 1e-12) / CUT
        xp = v * v * v * v * v
        a = -(P + 1) * (P + 2) / 2.0
        b = P * (P + 2)
        c = -P * (P + 1) / 2.0
        env = 1.0 / v + a * xp + b * xp * v + c * xp * v * v
        env = jnp.where(v < 1.0, env, jnp.zeros_like(v))
        rbf = env * jnp.sin(freq_ref[...] * v)
        ea = _silu(jnp.dot(rbf, rbfW_ref[...],
                           preferred_element_type=jnp.float32, precision=lax.Precision.HIGHEST) + rbfb_ref[...])
        ea_ref[...] = ea

    ea = ea_ref[...]
    ec = jnp.dot(ea, Wc_ref[0],
                 preferred_element_type=jnp.float32, precision=lax.Precision.HIGHEST) + xeb_ref[0]
    le = jnp.dot(ea, lin_ref[0], preferred_element_type=jnp.float32, precision=lax.Precision.HIGHEST)
    eci = lax.bitcast_convert_type(
        ec.astype(jnp.bfloat16).astype(jnp.float32), jnp.int32)
    lei = lax.bitcast_convert_type(
        le.astype(jnp.bfloat16).astype(jnp.float32), jnp.int32)
    el_ref[0] = lei | lax.shift_right_logical(eci, 16)


def _eale_tc(d2, freqp, rbfW8, rbf_b, Wc_all, lin_all, xe_b):
    grid = (E_PAD // BE, NL)
    return pl.pallas_call(
        _eale_body,
        grid=grid,
        in_specs=[
            pl.BlockSpec((BE, 1), lambda e, l: (e, 0)),
            pl.BlockSpec((1, 8), lambda e, l: (0, 0)),
            pl.BlockSpec((8, DIM), lambda e, l: (0, 0)),
            pl.BlockSpec((1, DIM), lambda e, l: (0, 0)),
            pl.BlockSpec((1, DIM, DIM), lambda e, l: (l, 0, 0)),
            pl.BlockSpec((1, DIM, DIM), lambda e, l: (l, 0, 0)),
            pl.BlockSpec((1, 1, DIM), lambda e, l: (l, 0, 0)),
        ],
        out_specs=pl.BlockSpec((1, BE, DIM), lambda e, l: (l, e, 0)),
        out_shape=jax.ShapeDtypeStruct((NL, E_PAD, DIM), jnp.int32),
        scratch_shapes=[pltpu.VMEM((BE, DIM), jnp.float32)],
    )(d2, freqp, rbfW8, rbf_b, Wc_all, lin_all, xe_b)


def _apart(h, hW_ref, hb_ref, Wa_ref, Wb_ref):
    hh = _silu(jnp.dot(h, hW_ref[0], preferred_element_type=jnp.float32, precision=lax.Precision.HIGHEST)
               + hb_ref[...])
    a = jnp.dot(hh, Wa_ref[0], preferred_element_type=jnp.float32, precision=lax.Precision.HIGHEST)
    b = jnp.dot(hh, Wb_ref[0], preferred_element_type=jnp.float32, precision=lax.Precision.HIGHEST)
    return hh, a, b


def _node0_body(xf_ref, embp_ref, hW_ref, hb_ref, Wa_ref, Wb_ref,
                h_ref, hh_ref, a_ref, b_ref):
    oh = (xf_ref[...] ==
          lax.broadcasted_iota(jnp.int32, (1, 8), 1).astype(jnp.float32))
    h = jnp.dot(oh.astype(jnp.float32), embp_ref[...],
                preferred_element_type=jnp.float32, precision=lax.Precision.HIGHEST)
    h_ref[...] = h
    hh, a, b = _apart(h, hW_ref, hb_ref, Wa_ref, Wb_ref)
    hh_ref[...] = hh
    a_ref[...] = a
    b_ref[...] = b


def _node0_tc(xf, embp, hW, hb, Wa, Wb):
    nspec = pl.BlockSpec((BN, DIM), lambda i: (i, 0))
    wspec = pl.BlockSpec((1, DIM, DIM), lambda i: (0, 0, 0))
    bspec = pl.BlockSpec((1, DIM), lambda i: (0, 0))
    return pl.pallas_call(
        _node0_body,
        grid=(N_PAD // BN,),
        in_specs=[
            pl.BlockSpec((BN, 1), lambda i: (i, 0)),
            pl.BlockSpec((8, DIM), lambda i: (0, 0)),
            wspec, bspec, wspec, wspec,
        ],
        out_specs=[nspec, nspec, nspec, nspec],
        out_shape=[jax.ShapeDtypeStruct((N_PAD, DIM), jnp.float32)] * 4,
    )(xf, embp, hW, hb, Wa, Wb)


def _bpart(p0, p1, hh, hres, y, resW_ref, resb_ref, mW_ref, mb_ref,
           oW_ref, ob_ref):
    t = p0 + p1 + hh + hres
    u = _silu(jnp.dot(t, resW_ref[0, 0, 0], preferred_element_type=jnp.float32, precision=lax.Precision.HIGHEST)
              + resb_ref[0, 0, 0][None])
    t = t + _silu(jnp.dot(u, resW_ref[0, 0, 1], preferred_element_type=jnp.float32, precision=lax.Precision.HIGHEST)
                  + resb_ref[0, 0, 1][None])
    h2 = _silu(jnp.dot(t, mW_ref[0], preferred_element_type=jnp.float32, precision=lax.Precision.HIGHEST)
               + mb_ref[...]) + hres
    u = _silu(jnp.dot(h2, resW_ref[0, 1, 0], preferred_element_type=jnp.float32, precision=lax.Precision.HIGHEST)
              + resb_ref[0, 1, 0][None])
    h2 = h2 + _silu(jnp.dot(u, resW_ref[0, 1, 1], preferred_element_type=jnp.float32, precision=lax.Precision.HIGHEST)
                    + resb_ref[0, 1, 1][None])
    u = _silu(jnp.dot(h2, resW_ref[0, 2, 0], preferred_element_type=jnp.float32, precision=lax.Precision.HIGHEST)
              + resb_ref[0, 2, 0][None])
    h2 = h2 + _silu(jnp.dot(u, resW_ref[0, 2, 1], preferred_element_type=jnp.float32, precision=lax.Precision.HIGHEST)
                    + resb_ref[0, 2, 1][None])
    ynew = y + jnp.dot(h2, oW_ref[0], preferred_element_type=jnp.float32, precision=lax.Precision.HIGHEST) \
        + ob_ref[...]
    return h2, ynew


def _nodeB_body(p0_ref, p1_ref, hh_ref, hres_ref, y_ref,
                resW_ref, resb_ref, mW_ref, mb_ref, oW_ref, ob_ref,
                h_ref, yout_ref):
    h2, ynew = _bpart(p0_ref[...], p1_ref[...], hh_ref[...], hres_ref[...],
                      y_ref[...], resW_ref, resb_ref, mW_ref, mb_ref,
                      oW_ref, ob_ref)
    h_ref[...] = h2
    yout_ref[...] = ynew


def _nodeBA_body(p0_ref, p1_ref, hh_ref, hres_ref, y_ref,
                 resW_ref, resb_ref, mW_ref, mb_ref, oW_ref, ob_ref,
                 hW_ref, hb_ref, Wa_ref, Wb_ref,
                 h_ref, yout_ref, hh2_ref, a_ref, b_ref):
    h2, ynew = _bpart(p0_ref[...], p1_ref[...], hh_ref[...], hres_ref[...],
                      y_ref[...], resW_ref, resb_ref, mW_ref, mb_ref,
                      oW_ref, ob_ref)
    h_ref[...] = h2
    yout_ref[...] = ynew
    hh, a, b = _apart(h2, hW_ref, hb_ref, Wa_ref, Wb_ref)
    hh2_ref[...] = hh
    a_ref[...] = a
    b_ref[...] = b


def _node_tc(p0, p1, hh, hres, y, resW, resb, mW, mb, oW, ob,
             nxt=None):
    nspec = pl.BlockSpec((BN, DIM), lambda i: (i, 0))
    yspec = pl.BlockSpec((BN, 1), lambda i: (i, 0))
    wspec = pl.BlockSpec((1, DIM, DIM), lambda i: (0, 0, 0))
    bspec = pl.BlockSpec((1, DIM), lambda i: (0, 0))
    in_specs = [
        nspec, nspec, nspec, nspec, yspec,
        pl.BlockSpec((1, 3, 2, DIM, DIM), lambda i: (0, 0, 0, 0, 0)),
        pl.BlockSpec((1, 3, 2, DIM), lambda i: (0, 0, 0, 0)),
        wspec, bspec,
        pl.BlockSpec((1, DIM, 1), lambda i: (0, 0, 0)),
        pl.BlockSpec((1, 1), lambda i: (0, 0)),
    ]
    args = [p0, p1, hh, hres, y, resW, resb, mW, mb, oW, ob]
    if nxt is None:
        return pl.pallas_call(
            _nodeB_body,
            grid=(N_PAD // BN,),
            in_specs=in_specs,
            out_specs=[nspec, yspec],
            out_shape=[jax.ShapeDtypeStruct((N_PAD, DIM), jnp.float32),
                       jax.ShapeDtypeStruct((N_PAD, 1), jnp.float32)],
        )(*args)
    hW, hb, Wa, Wb = nxt
    return pl.pallas_call(
        _nodeBA_body,
        grid=(N_PAD // BN,),
        in_specs=in_specs + [wspec, bspec, wspec, wspec],
        out_specs=[nspec, yspec, nspec, nspec, nspec],
        out_shape=[jax.ShapeDtypeStruct((N_PAD, DIM), jnp.float32),
                   jax.ShapeDtypeStruct((N_PAD, 1), jnp.float32),
                   jax.ShapeDtypeStruct((N_PAD, DIM), jnp.float32),
                   jax.ShapeDtypeStruct((N_PAD, DIM), jnp.float32),
                   jax.ShapeDtypeStruct((N_PAD, DIM), jnp.float32)],
    )(*args, hW, hb, Wa, Wb)


def _pool_body(y_ref, bf_ref, out_ref):
    i = pl.program_id(0)

    @pl.when(i == 0)
    def _():
        out_ref[...] = jnp.zeros_like(out_ref)

    oh = (bf_ref[...] ==
          lax.broadcasted_iota(jnp.int32, (1, NG), 1).astype(jnp.float32))
    out_ref[...] += jnp.sum(oh.astype(jnp.float32) * y_ref[...], axis=0,
                            keepdims=True)


def _pool_tc(y, bf):
    return pl.pallas_call(
        _pool_body,
        grid=(N_PAD // BN,),
        in_specs=[pl.BlockSpec((BN, 1), lambda i: (i, 0)),
                  pl.BlockSpec((BN, 1), lambda i: (i, 0))],
        out_specs=pl.BlockSpec((1, NG), lambda i: (0, 0)),
        out_shape=jax.ShapeDtypeStruct((1, NG), jnp.float32),
    )(y, bf)


def kernel(x, pos, edge_index, batch, emb, freq, rbf_W, rbf_b, h_W, h_b,
           xe_W, xe_b, lin_W, res_W, res_b, mlp_W, mlp_b, out_W, out_b):
    f32 = jnp.float32
    rows = jnp.concatenate(
        [edge_index[0].astype(jnp.int32),
         jnp.full((E_PAD - E,), N, jnp.int32)])
    cols = jnp.concatenate(
        [edge_index[1].astype(jnp.int32),
         jnp.full((E_PAD - E,), N, jnp.int32)])
    posp = jnp.concatenate([pos.astype(f32),
                            jnp.zeros((N_PAD - N, 3), f32)], axis=0)
    px, py, pz = posp[:, 0], posp[:, 1], posp[:, 2]
    xf = jnp.concatenate([x.astype(f32), jnp.zeros((N_PAD - N,), f32)]
                         ).reshape(N_PAD, 1)
    bf = jnp.concatenate([batch.astype(f32),
                          jnp.full((N_PAD - N,), float(NG), f32)]
                         ).reshape(N_PAD, 1)

    embp = jnp.concatenate([emb, jnp.zeros((3, DIM), f32)], axis=0)
    freqp = jnp.concatenate([freq, jnp.zeros((2,), f32)]).reshape(1, 8)
    rbfW8 = jnp.concatenate([rbf_W, jnp.zeros((2, DIM), f32)], axis=0)
    rbfb2 = rbf_b.reshape(1, DIM)
    Wa = xe_W[:, :DIM]
    Wb = xe_W[:, DIM:2 * DIM]
    Wc = xe_W[:, 2 * DIM:]
    hb2 = h_b.reshape(NL, 1, DIM)
    mb2 = mlp_b.reshape(NL, 1, DIM)
    ob2 = out_b.reshape(NL, 1, 1)

    d2 = _d2_sc(px, py, pz, rows, cols)
    el_all = _eale_tc(d2.reshape(E_PAD, 1), freqp, rbfW8, rbfb2,
                      Wc, lin_W, xe_b.reshape(NL, 1, DIM))
    el_i32 = el_all

    h, hh, a, b = _node0_tc(xf, embp, h_W[0:1], hb2[0], Wa[0:1], Wb[0:1])
    y = jnp.zeros((N_PAD, 1), f32)
    for l in range(NL):
        p0, p1 = _edge_sc(a, b, el_i32[l], rows, cols)
        nxt = None if l == NL - 1 else (h_W[l + 1:l + 2], hb2[l + 1],
                                        Wa[l + 1:l + 2], Wb[l + 1:l + 2])
        outs = _node_tc(p0, p1, hh, h, y,
                        res_W[l:l + 1], res_b[l:l + 1], mlp_W[l:l + 1],
                        mb2[l], out_W[l:l + 1], ob2[l], nxt=nxt)
        if l == NL - 1:
            h, y = outs
        else:
            h, y, hh, a, b = outs

    pooled = _pool_tc(y, bf)
    return pooled.reshape(NG, 1)
